# baseline (speedup 1.0000x reference)
import jax
import jax.numpy as jnp
from jax.experimental import pallas as pl
from jax.experimental.pallas import tpu as pltpu

_N_CHUNKS = 16


def kernel(x, table):
    del x
    max_len, emb_dim = table.shape
    rows_per_chunk = max_len // _N_CHUNKS

    def body(in_hbm, out_hbm, sem):
        copies = [
            pltpu.make_async_copy(
                in_hbm.at[pl.ds(c * rows_per_chunk, rows_per_chunk)],
                out_hbm.at[pl.ds(c * rows_per_chunk, rows_per_chunk)],
                sem,
            )
            for c in range(_N_CHUNKS)
        ]
        for cp in copies:
            cp.start()
        for cp in copies:
            cp.wait()

    out = pl.pallas_call(
        body,
        in_specs=[pl.BlockSpec(memory_space=pltpu.MemorySpace.HBM)],
        out_specs=pl.BlockSpec(memory_space=pltpu.MemorySpace.HBM),
        out_shape=jax.ShapeDtypeStruct((max_len, emb_dim), table.dtype),
        scratch_shapes=[pltpu.SemaphoreType.DMA],
    )(table)
    return out[None]


# speedup vs baseline: 33.9447x; 33.9447x over previous
import jax
import jax.numpy as jnp
from jax.experimental import pallas as pl
from jax.experimental.pallas import tpu as pltpu

_CHUNK_ROWS = 512


def kernel(x, table):
    del x
    max_len, emb_dim = table.shape
    nch = max_len // _CHUNK_ROWS

    def body(in_hbm, out_hbm, buf, si0, si1, so0, so1):
        sin = (si0, si1)
        sout = (so0, so1)

        def cin(i):
            return pltpu.make_async_copy(
                in_hbm.at[pl.ds(i * _CHUNK_ROWS, _CHUNK_ROWS)],
                buf.at[i % 2],
                sin[i % 2],
            )

        def cout(i):
            return pltpu.make_async_copy(
                buf.at[i % 2],
                out_hbm.at[pl.ds(i * _CHUNK_ROWS, _CHUNK_ROWS)],
                sout[i % 2],
            )

        cin(0).start()
        for i in range(nch):
            if i + 1 < nch:
                if i >= 1:
                    cout(i - 1).wait()
                cin(i + 1).start()
            cin(i).wait()
            cout(i).start()
        if nch >= 2:
            cout(nch - 2).wait()
        cout(nch - 1).wait()

    out = pl.pallas_call(
        body,
        in_specs=[pl.BlockSpec(memory_space=pltpu.MemorySpace.HBM)],
        out_specs=pl.BlockSpec(memory_space=pltpu.MemorySpace.HBM),
        out_shape=jax.ShapeDtypeStruct((max_len, emb_dim), table.dtype),
        scratch_shapes=[
            pltpu.VMEM((2, _CHUNK_ROWS, emb_dim), table.dtype),
            pltpu.SemaphoreType.DMA,
            pltpu.SemaphoreType.DMA,
            pltpu.SemaphoreType.DMA,
            pltpu.SemaphoreType.DMA,
        ],
    )(table)
    return out[None]


# speedup vs baseline: 36.6770x; 1.0805x over previous
import jax
import jax.numpy as jnp
from jax.experimental import pallas as pl
from jax.experimental.pallas import tpu as pltpu

_CHUNK_ROWS = 128
_NBUF = 8
_PREFETCH = 4


def kernel(x, table):
    del x
    max_len, emb_dim = table.shape
    nch = max_len // _CHUNK_ROWS

    def body(in_hbm, out_hbm, buf, *sems):
        sin = sems[:_NBUF]
        sout = sems[_NBUF:]

        def cin(i):
            return pltpu.make_async_copy(
                in_hbm.at[pl.ds(i * _CHUNK_ROWS, _CHUNK_ROWS)],
                buf.at[i % _NBUF],
                sin[i % _NBUF],
            )

        def cout(i):
            return pltpu.make_async_copy(
                buf.at[i % _NBUF],
                out_hbm.at[pl.ds(i * _CHUNK_ROWS, _CHUNK_ROWS)],
                sout[i % _NBUF],
            )

        for i in range(min(_PREFETCH, nch)):
            cin(i).start()
        for i in range(nch):
            cin(i).wait()
            cout(i).start()
            j = i + _PREFETCH
            if j < nch:
                if j >= _NBUF:
                    cout(j - _NBUF).wait()
                cin(j).start()
        for i in range(max(nch - _NBUF, 0), nch):
            cout(i).wait()

    out = pl.pallas_call(
        body,
        in_specs=[pl.BlockSpec(memory_space=pltpu.MemorySpace.HBM)],
        out_specs=pl.BlockSpec(memory_space=pltpu.MemorySpace.HBM),
        out_shape=jax.ShapeDtypeStruct((max_len, emb_dim), table.dtype),
        scratch_shapes=[pltpu.VMEM((_NBUF, _CHUNK_ROWS, emb_dim), table.dtype)]
        + [pltpu.SemaphoreType.DMA] * (2 * _NBUF),
    )(table)
    return out[None]


# speedup vs baseline: 46.7525x; 1.2747x over previous
import jax
import jax.numpy as jnp
from jax.experimental import pallas as pl
from jax.experimental.pallas import tpu as pltpu

_CHUNK_ROWS = 1024
_NBUF = 4
_PREFETCH = 2


def kernel(x, table):
    del x
    max_len, emb_dim = table.shape
    nch = max_len // _CHUNK_ROWS

    def body(in_hbm, out_hbm, buf, *sems):
        sin = sems[:_NBUF]
        sout = sems[_NBUF:]

        def cin(i):
            return pltpu.make_async_copy(
                in_hbm.at[pl.ds(i * _CHUNK_ROWS, _CHUNK_ROWS)],
                buf.at[i % _NBUF],
                sin[i % _NBUF],
            )

        def cout(i):
            return pltpu.make_async_copy(
                buf.at[i % _NBUF],
                out_hbm.at[pl.ds(i * _CHUNK_ROWS, _CHUNK_ROWS)],
                sout[i % _NBUF],
            )

        for i in range(min(_PREFETCH, nch)):
            cin(i).start()
        for i in range(nch):
            cin(i).wait()
            cout(i).start()
            j = i + _PREFETCH
            if j < nch:
                if j >= _NBUF:
                    cout(j - _NBUF).wait()
                cin(j).start()
        for i in range(max(nch - _NBUF, 0), nch):
            cout(i).wait()

    out = pl.pallas_call(
        body,
        in_specs=[pl.BlockSpec(memory_space=pltpu.MemorySpace.HBM)],
        out_specs=pl.BlockSpec(memory_space=pltpu.MemorySpace.HBM),
        out_shape=jax.ShapeDtypeStruct((max_len, emb_dim), table.dtype),
        scratch_shapes=[pltpu.VMEM((_NBUF, _CHUNK_ROWS, emb_dim), table.dtype)]
        + [pltpu.SemaphoreType.DMA] * (2 * _NBUF),
    )(table)
    return out[None]


# speedup vs baseline: 48.9185x; 1.0463x over previous
import jax
import jax.numpy as jnp
from jax.experimental import pallas as pl
from jax.experimental.pallas import tpu as pltpu

_CHUNK_ROWS = 2048
_NBUF = 4
_PREFETCH = 2


def kernel(x, table):
    del x
    max_len, emb_dim = table.shape
    nch = max_len // _CHUNK_ROWS

    def body(in_hbm, out_hbm, buf, *sems):
        sin = sems[:_NBUF]
        sout = sems[_NBUF:]

        def cin(i):
            return pltpu.make_async_copy(
                in_hbm.at[pl.ds(i * _CHUNK_ROWS, _CHUNK_ROWS)],
                buf.at[i % _NBUF],
                sin[i % _NBUF],
            )

        def cout(i):
            return pltpu.make_async_copy(
                buf.at[i % _NBUF],
                out_hbm.at[pl.ds(i * _CHUNK_ROWS, _CHUNK_ROWS)],
                sout[i % _NBUF],
            )

        for i in range(min(_PREFETCH, nch)):
            cin(i).start()
        for i in range(nch):
            cin(i).wait()
            cout(i).start()
            j = i + _PREFETCH
            if j < nch:
                if j >= _NBUF:
                    cout(j - _NBUF).wait()
                cin(j).start()
        for i in range(max(nch - _NBUF, 0), nch):
            cout(i).wait()

    out = pl.pallas_call(
        body,
        in_specs=[pl.BlockSpec(memory_space=pltpu.MemorySpace.HBM)],
        out_specs=pl.BlockSpec(memory_space=pltpu.MemorySpace.HBM),
        out_shape=jax.ShapeDtypeStruct((max_len, emb_dim), table.dtype),
        scratch_shapes=[pltpu.VMEM((_NBUF, _CHUNK_ROWS, emb_dim), table.dtype)]
        + [pltpu.SemaphoreType.DMA] * (2 * _NBUF),
    )(table)
    return out[None]
